# baseline (speedup 1.0000x reference)
import functools

import jax
import jax.numpy as jnp
from jax import lax
from jax.experimental import pallas as pl
from jax.experimental.pallas import tpu as pltpu
from jax.experimental.pallas import tpu_sc as plsc

NC = 2
NS = 16
NW = NC * NS
K = 128
G = 4


@functools.lru_cache(maxsize=None)
def _make_segment_sum_sc(n, d, ch):
    rows_per_tile = -(-(n + 1) // (NS * K)) * K
    agg_rows = rows_per_tile * NS
    mesh = plsc.VectorSubcoreMesh(core_axis_name="c", subcore_axis_name="s",
                                  num_cores=NC, num_subcores=NS)

    @functools.partial(
        pl.kernel,
        out_type=jax.ShapeDtypeStruct((NC, agg_rows, d), jnp.bfloat16),
        mesh=mesh,
        scratch_types=[
            pltpu.VMEM((ch, K), jnp.int32),
            pltpu.VMEM((ch, K), jnp.int32),
            pltpu.VMEM((G, K, d), jnp.bfloat16),
            pltpu.VMEM((K, d), jnp.bfloat16),
            pltpu.VMEM_SHARED((agg_rows, d), jnp.bfloat16),
            pltpu.SemaphoreType.DMA,
            pltpu.SemaphoreType.DMA,
            pltpu.SemaphoreType.DMA,
            pltpu.SemaphoreType.DMA,
        ],
        compiler_params=pltpu.CompilerParams(use_tc_tiling_on_sc=False),
    )
    def seg_sum(table, srcp, dstp, out, src_v, dst_v, rows_v, zero_v, agg_sh,
                sga, sgb, ssa, ssb):
        c = lax.axis_index("c")
        s = lax.axis_index("s")
        wid = s * NC + c

        GS = G // 2

        def _fire_g(buf0, chunk0, sem):
            for g in range(GS):
                pltpu.async_copy(table.at[src_v.at[chunk0 + g]],
                                 rows_v.at[buf0 + g], sem)

        def _drain_g(buf0, sem):
            for g in range(GS):
                pltpu.make_async_copy(table.at[src_v.at[0]],
                                      rows_v.at[buf0 + g], sem).wait()

        def _fire_s(buf0, chunk0, sem):
            for g in range(GS):
                pltpu.async_copy(rows_v.at[buf0 + g],
                                 agg_sh.at[dst_v.at[chunk0 + g]], sem,
                                 add=True)

        def _drain_s(buf0, sem):
            for g in range(GS):
                pltpu.make_async_copy(rows_v.at[buf0 + g],
                                      agg_sh.at[dst_v.at[0]], sem).wait()

        dsteps = ch // G
        pltpu.sync_copy(srcp.at[wid], src_v)
        pltpu.sync_copy(dstp.at[wid], dst_v)
        _fire_g(0, 0, sga)

        def _zero_row(i, _):
            for j in range(d // 32):
                zero_v[i, pl.ds(j * 32, 32)] = jnp.zeros((32,), jnp.bfloat16)
            return 0
        lax.fori_loop(0, K, _zero_row, 0)
        for t in range(rows_per_tile // K):
            pltpu.sync_copy(zero_v,
                            agg_sh.at[pl.ds(s * rows_per_tile + t * K, K)])
        plsc.subcore_barrier()

        def _dstep(ds, _):
            c0 = ds * G
            _drain_g(0, sga)
            _fire_s(0, c0, ssa)

            @pl.when(ds > 0)
            def _():
                _drain_s(2, ssb)

            _fire_g(2, c0 + GS, sgb)
            _drain_g(2, sgb)
            _fire_s(2, c0 + GS, ssb)
            _drain_s(0, ssa)

            @pl.when(ds < dsteps - 1)
            def _():
                _fire_g(0, c0 + G, sga)
            return 0
        lax.fori_loop(0, dsteps, _dstep, 0)
        _drain_s(2, ssb)

        plsc.subcore_barrier()
        pltpu.sync_copy(agg_sh.at[pl.ds(s * rows_per_tile, rows_per_tile)],
                        out.at[c, pl.ds(s * rows_per_tile, rows_per_tile)])

    return seg_sum


def _unpack_bf16_words(p):
    lo = lax.bitcast_convert_type(jnp.left_shift(p, 16), jnp.float32)
    hi = lax.bitcast_convert_type(
        jnp.bitwise_and(p, jnp.int32(-65536)), jnp.float32)
    return lo, hi


def _round_bf16_bits(x):
    u = lax.bitcast_convert_type(x, jnp.int32)
    return u + 0x7FFF + jnp.bitwise_and(jnp.right_shift(u, 16), 1)


def _layer_tc(cur_ref, p_ref, w_ref, b_ref, g_ref, be_ref,
              out_ref, outpk_ref, sin_ref, sout_ref):
    n = cur_ref.shape[0]
    nh = n // 2
    dh = cur_ref.shape[1] // 2
    lo0, hi0 = _unpack_bf16_words(p_ref[0, :nh, :])
    lo1, hi1 = _unpack_bf16_words(p_ref[1, :nh, :])
    lo = lo0 + lo1
    hi = hi0 + hi1
    agg = jnp.concatenate(
        [jnp.concatenate([lo[:, :dh], hi[:, :dh]], axis=1),
         jnp.concatenate([lo[:, dh:], hi[:, dh:]], axis=1)],
        axis=0)
    cur = cur_ref[...]
    r = cur + agg
    z = jnp.dot(r, w_ref[...], preferred_element_type=jnp.float32) + b_ref[...]
    m = jnp.mean(z, axis=0, keepdims=True)
    v = jnp.mean((z - m) ** 2, axis=0, keepdims=True)
    zn = (z - m) * lax.rsqrt(v + 1e-5) * g_ref[...] + be_ref[...]
    outv = jnp.maximum(zn, 0.0)
    out_ref[...] = outv
    ta = _round_bf16_bits(outv[:, :dh])
    tb = _round_bf16_bits(outv[:, dh:])
    words = jnp.bitwise_or(
        jnp.bitwise_and(tb, jnp.int32(-65536)),
        jnp.bitwise_and(jnp.right_shift(ta, 16), 0xFFFF))
    outpk_ref[...] = jnp.concatenate([words[:nh], words[nh:]], axis=1)
    sin_ref[...] = jnp.sum(cur, axis=0, keepdims=True)
    sout_ref[...] = jnp.sum(outv, axis=0, keepdims=True)


@functools.lru_cache(maxsize=None)
def _make_layer_tc(n, d):
    return pl.pallas_call(
        _layer_tc,
        out_shape=[jax.ShapeDtypeStruct((n, d), jnp.float32),
                   jax.ShapeDtypeStruct((n // 2, d), jnp.int32),
                   jax.ShapeDtypeStruct((1, d), jnp.float32),
                   jax.ShapeDtypeStruct((1, d), jnp.float32)],
    )


def _score_tc(pool_ref, wp_ref, bp_ref, out_ref):
    acc = jnp.sum(bp_ref[...], axis=0, keepdims=True)
    for i in range(pool_ref.shape[0]):
        acc = acc + jnp.dot(pool_ref[pl.ds(i, 1), :], wp_ref[i],
                            preferred_element_type=jnp.float32)
    out_ref[...] = acc


@functools.lru_cache(maxsize=None)
def _make_score_tc(d):
    return pl.pallas_call(
        _score_tc,
        out_shape=jax.ShapeDtypeStruct((1, d), jnp.float32),
    )


def kernel(h, edge_index, params):
    n, d = h.shape
    e = edge_index.shape[1]
    ch = -(-e // (NW * K * G)) * G
    epad = NW * ch * K

    rows_per_tile = -(-(n + 1) // (NS * K)) * K
    agg_rows = rows_per_tile * NS

    nh = n // 2
    dh = d // 2
    src = edge_index[0].astype(jnp.int32)
    dst = edge_index[1].astype(jnp.int32)
    npad = epad - e
    pad_src = jnp.arange(npad, dtype=jnp.int32) % n
    pad_dst = n + jnp.arange(npad, dtype=jnp.int32) % (agg_rows - n)

    def sigma(t):
        return jnp.where(t < nh, 2 * t,
                         jnp.where(t < n, 2 * (t - nh) + 1, t))

    srcp = sigma(jnp.concatenate([src, pad_src])).reshape(NW, ch, K)
    dstp = sigma(jnp.concatenate([dst, pad_dst])).reshape(NW, ch, K)

    seg_sum = _make_segment_sum_sc(n, d, ch)
    layer_call = _make_layer_tc(n, d)

    def table_of(pk):
        return lax.bitcast_convert_type(pk, jnp.bfloat16).reshape(n, d)

    def i32_view(part):
        return lax.bitcast_convert_type(
            part.reshape(NC, agg_rows, d // 2, 2), jnp.int32).reshape(
                NC, agg_rows // 2, d)

    hb = h.astype(jnp.bfloat16)
    ha = lax.bitcast_convert_type(hb[:, :dh], jnp.uint16).astype(jnp.int32)
    hc = lax.bitcast_convert_type(hb[:, dh:], jnp.uint16).astype(jnp.int32)
    hw = jnp.bitwise_or(jnp.left_shift(hc, 16), ha)
    curpk = jnp.concatenate([hw[:nh], hw[nh:]], axis=1)

    cur = h
    pools = []
    for i, lay in enumerate(params['layers']):
        part = seg_sum(table_of(curpk), srcp, dstp)
        cur, curpk, sin, sout = layer_call(cur, i32_view(part), lay['W'],
                                           lay['b'].reshape(1, d),
                                           lay['g'].reshape(1, d),
                                           lay['be'].reshape(1, d))
        if i == 0:
            pools.append(sin)
        pools.append(sout)

    pool = jnp.concatenate(pools, axis=0)
    wp = jnp.stack([p['W'] for p in params['pred']])
    bp = jnp.stack([p['b'] for p in params['pred']])
    return _make_score_tc(d)(pool, wp, bp)


# speedup vs baseline: 4.0079x; 4.0079x over previous
import functools

import jax
import jax.numpy as jnp
from jax import lax
from jax.experimental import pallas as pl
from jax.experimental.pallas import tpu as pltpu
from jax.experimental.pallas import tpu_sc as plsc

NC = 2
NS = 16
NW = NC * NS
K = 128
G = 4


@functools.lru_cache(maxsize=None)
def _make_segment_sum_sc(n, d, ch):
    rows_per_tile = -(-(n + 1) // (NS * K)) * K
    agg_rows = rows_per_tile * NS
    mesh = plsc.VectorSubcoreMesh(core_axis_name="c", subcore_axis_name="s",
                                  num_cores=NC, num_subcores=NS)

    @functools.partial(
        pl.kernel,
        out_type=jax.ShapeDtypeStruct((NC, agg_rows, d), jnp.bfloat16),
        mesh=mesh,
        scratch_types=[
            pltpu.VMEM((ch, K), jnp.int32),
            pltpu.VMEM((ch, K), jnp.int32),
            pltpu.VMEM((G, K, d), jnp.bfloat16),
            pltpu.VMEM((K, d), jnp.bfloat16),
            pltpu.VMEM_SHARED((agg_rows, d), jnp.bfloat16),
            pltpu.SemaphoreType.DMA,
            pltpu.SemaphoreType.DMA,
            pltpu.SemaphoreType.DMA,
            pltpu.SemaphoreType.DMA,
        ],
        compiler_params=pltpu.CompilerParams(use_tc_tiling_on_sc=False),
    )
    def seg_sum(table, srcp, dstp, out, src_v, dst_v, rows_v, zero_v, agg_sh,
                sga, sgb, ssa, ssb):
        c = lax.axis_index("c")
        s = lax.axis_index("s")
        wid = s * NC + c

        GS = G // 2

        def _fire_g(buf0, chunk0, sem):
            for g in range(GS):
                pltpu.async_copy(table.at[src_v.at[chunk0 + g]],
                                 rows_v.at[buf0 + g], sem)

        def _drain_g(buf0, sem):
            for g in range(GS):
                pltpu.make_async_copy(table.at[src_v.at[0]],
                                      rows_v.at[buf0 + g], sem).wait()

        def _fire_s(buf0, chunk0, sem):
            for g in range(GS):
                pltpu.async_copy(rows_v.at[buf0 + g],
                                 agg_sh.at[dst_v.at[chunk0 + g]], sem,
                                 add=True)

        def _drain_s(buf0, sem):
            for g in range(GS):
                pltpu.make_async_copy(rows_v.at[buf0 + g],
                                      agg_sh.at[dst_v.at[0]], sem).wait()

        dsteps = ch // G
        pltpu.sync_copy(srcp.at[wid], src_v)
        pltpu.sync_copy(dstp.at[wid], dst_v)
        _fire_g(0, 0, sga)

        def _zero_row(i, _):
            for j in range(d // 32):
                zero_v[i, pl.ds(j * 32, 32)] = jnp.zeros((32,), jnp.bfloat16)
            return 0
        lax.fori_loop(0, K, _zero_row, 0)
        for t in range(rows_per_tile // K):
            pltpu.sync_copy(zero_v,
                            agg_sh.at[pl.ds(s * rows_per_tile + t * K, K)])
        plsc.subcore_barrier()

        def _dstep(ds, _):
            c0 = ds * G
            _drain_g(0, sga)
            _fire_s(0, c0, ssa)

            @pl.when(ds > 0)
            def _():
                _drain_s(2, ssb)

            _fire_g(2, c0 + GS, sgb)
            _drain_g(2, sgb)
            _fire_s(2, c0 + GS, ssb)
            _drain_s(0, ssa)

            @pl.when(ds < dsteps - 1)
            def _():
                _fire_g(0, c0 + G, sga)
            return 0
        lax.fori_loop(0, dsteps, _dstep, 0)
        _drain_s(2, ssb)

        plsc.subcore_barrier()
        pltpu.sync_copy(agg_sh.at[pl.ds(s * rows_per_tile, rows_per_tile)],
                        out.at[c, pl.ds(s * rows_per_tile, rows_per_tile)])

    return seg_sum


def _layer_tc(cur_ref, p_ref, w_ref, b_ref, g_ref, be_ref,
              out_ref, outbf_ref, sin_ref, sout_ref):
    n = cur_ref.shape[0]
    cur = cur_ref[...]
    agg = (p_ref[0, :n, :].astype(jnp.float32)
           + p_ref[1, :n, :].astype(jnp.float32))
    r = cur + agg
    z = jnp.dot(r, w_ref[...], preferred_element_type=jnp.float32) + b_ref[...]
    m = jnp.mean(z, axis=0, keepdims=True)
    v = jnp.mean((z - m) ** 2, axis=0, keepdims=True)
    zn = (z - m) * lax.rsqrt(v + 1e-5) * g_ref[...] + be_ref[...]
    outv = jnp.maximum(zn, 0.0)
    out_ref[...] = outv
    outbf_ref[...] = outv.astype(jnp.bfloat16)
    sin_ref[...] = jnp.sum(cur, axis=0, keepdims=True)
    sout_ref[...] = jnp.sum(outv, axis=0, keepdims=True)


@functools.lru_cache(maxsize=None)
def _make_layer_tc(n, d):
    return pl.pallas_call(
        _layer_tc,
        out_shape=[jax.ShapeDtypeStruct((n, d), jnp.float32),
                   jax.ShapeDtypeStruct((n, d), jnp.bfloat16),
                   jax.ShapeDtypeStruct((1, d), jnp.float32),
                   jax.ShapeDtypeStruct((1, d), jnp.float32)],
    )


def _score_tc(pool_ref, wp_ref, bp_ref, out_ref):
    acc = jnp.sum(bp_ref[...], axis=0, keepdims=True)
    for i in range(pool_ref.shape[0]):
        acc = acc + jnp.dot(pool_ref[pl.ds(i, 1), :], wp_ref[i],
                            preferred_element_type=jnp.float32)
    out_ref[...] = acc


@functools.lru_cache(maxsize=None)
def _make_score_tc(d):
    return pl.pallas_call(
        _score_tc,
        out_shape=jax.ShapeDtypeStruct((1, d), jnp.float32),
    )


def kernel(h, edge_index, params):
    n, d = h.shape
    e = edge_index.shape[1]
    ch = -(-e // (NW * K * G)) * G
    epad = NW * ch * K

    rows_per_tile = -(-(n + 1) // (NS * K)) * K
    agg_rows = rows_per_tile * NS

    src = edge_index[0].astype(jnp.int32)
    dst = edge_index[1].astype(jnp.int32)
    npad = epad - e
    pad_src = jnp.arange(npad, dtype=jnp.int32) % n
    pad_dst = n + jnp.arange(npad, dtype=jnp.int32) % (agg_rows - n)
    srcp = jnp.concatenate([src, pad_src]).reshape(NW, ch, K)
    dstp = jnp.concatenate([dst, pad_dst]).reshape(NW, ch, K)

    seg_sum = _make_segment_sum_sc(n, d, ch)
    layer_call = _make_layer_tc(n, d)

    cur = h
    curbf = h.astype(jnp.bfloat16)
    pools = []
    for i, lay in enumerate(params['layers']):
        part = seg_sum(curbf, srcp, dstp)
        cur, curbf, sin, sout = layer_call(cur, part, lay['W'],
                                           lay['b'].reshape(1, d),
                                           lay['g'].reshape(1, d),
                                           lay['be'].reshape(1, d))
        if i == 0:
            pools.append(sin)
        pools.append(sout)

    pool = jnp.concatenate(pools, axis=0)
    wp = jnp.stack([p['W'] for p in params['pred']])
    bp = jnp.stack([p['b'] for p in params['pred']])
    return _make_score_tc(d)(pool, wp, bp)
